# Initial kernel scaffold; baseline (speedup 1.0000x reference)
import jax
import jax.numpy as jnp
from jax.experimental import pallas as pl


def kernel(x, edge_index, batch_index, alpha, W0, as0, ad0, b0, Wr0, W1, as1, ad1, b1, Wr1, W2, as2, ad2, b2, Wr2, W3, as3, ad3, b3, Wr3, Wout, bout):
    raise NotImplementedError("write your pallas kernel here")


# speedup vs baseline: 1.0000x; 1.0000x over previous
import jax
import jax.numpy as jnp
from jax.experimental import pallas as pl


def _gat_layer(x, edge_index, W, att_s, att_d, b, Wr):
    src = edge_index[0]
    dst = edge_index[1]
    n = x.shape[0]
    xp = x @ W
    a_s = (xp * att_s).sum(-1)
    a_d = (xp * att_d).sum(-1)
    e = jax.nn.leaky_relu(a_s[src] + a_d[dst], 0.2)
    e_max = jax.ops.segment_max(e, dst, num_segments=n)
    e_max = jnp.where(jnp.isfinite(e_max), e_max, 0.0)
    ex = jnp.exp(e - e_max[dst])
    denom = jax.ops.segment_sum(ex, dst, num_segments=n)
    attn = ex / (denom[dst] + 1e-16)
    out = jax.ops.segment_sum(attn[:, None] * xp[src], dst, num_segments=n)
    return out + x @ Wr + b


def kernel(x, edge_index, batch_index, alpha, W0, as0, ad0, b0, Wr0, W1, as1, ad1, b1, Wr1, W2, as2, ad2, b2, Wr2, W3, as3, ad3, b3, Wr3, Wout, bout):
    B = 128
    h = _gat_layer(x, edge_index, W0, as0, ad0, b0, Wr0)
    h = jax.nn.leaky_relu(h, 0.01)
    h = _gat_layer(h, edge_index, W1, as1, ad1, b1, Wr1)
    h = jax.nn.leaky_relu(h, 0.01)
    h = _gat_layer(h, edge_index, W2, as2, ad2, b2, Wr2)
    h = jax.nn.leaky_relu(h, 0.01)
    h = _gat_layer(h, edge_index, W3, as3, ad3, b3, Wr3)
    h = alpha * h
    h = jax.ops.segment_max(h, batch_index, num_segments=B)
    h = jnp.where(jnp.isfinite(h), h, 0.0)
    h = jax.nn.leaky_relu(h, 0.01)
    return h @ Wout + bout


# speedup vs baseline: 17.0816x; 17.0814x over previous
import functools

import jax
import jax.numpy as jnp
from jax import lax
from jax.experimental import pallas as pl
from jax.experimental.pallas import tpu as pltpu
from jax.experimental.pallas import tpu_sc as plsc

N = 50000
D = 128
F_IN = 93
NRANGE = 12544
TRASH = NRANGE
ACCROWS = NRANGE + 16
E = 800000
K = 64
RPT = NRANGE // 16
NADP = 50176
EPAD = E + 16 * K
POOL_RPT = 1568
NEG = -3.0e38


def _mm_body(h_ref, wc_ref, a_ref, b_ref, xp_ref, r_ref, av_ref):
    hb = h_ref[...]
    y = jnp.dot(hb, wc_ref[...], preferred_element_type=jnp.float32)
    xp = y[:, :D]
    xp_ref[...] = xp
    av_ref[...] = jnp.dot(xp, a_ref[...], preferred_element_type=jnp.float32)
    r_ref[...] = y[:, D:2 * D] + b_ref[...]


def _tc_layer(h, W, Wr, att_s, att_d, b):
    wc = jnp.concatenate([W, Wr], axis=1)
    amat = jnp.stack([att_s, att_d], axis=1)
    bn = 2000
    return pl.pallas_call(
        _mm_body,
        grid=(N // bn,),
        in_specs=[pl.BlockSpec((bn, D), lambda i: (i, 0)),
                  pl.BlockSpec((D, 2 * D), lambda i: (0, 0)),
                  pl.BlockSpec((D, 2), lambda i: (0, 0)),
                  pl.BlockSpec((1, D), lambda i: (0, 0))],
        out_specs=[pl.BlockSpec((bn, D), lambda i: (i, 0)),
                   pl.BlockSpec((bn, D), lambda i: (i, 0)),
                   pl.BlockSpec((bn, 2), lambda i: (i, 0))],
        out_shape=[jax.ShapeDtypeStruct((N, D), jnp.float32),
                   jax.ShapeDtypeStruct((N, D), jnp.float32),
                   jax.ShapeDtypeStruct((N, 2), jnp.float32)],
    )(h, wc, amat, b.reshape(1, D))


def _edge_body(do_leaky, xp, bsrc, bdst, ast, adt, r_h, meta_h, out,
               accum, denacc, srcv, dstv, locv, wv, sbuf, adbuf, dbuf,
               rows, rbuf, obuf, meta_v, sem, sem2, sem3):
    c = lax.axis_index("c")
    s = lax.axis_index("s")
    pltpu.sync_copy(meta_h, meta_v)

    def one_pass(p, carry):
        g = 2 * c + p
        base = g * NRANGE
        for j in range(16):
            for cc in range(8):
                rows[j, pl.ds(cc * 16, 16)] = jnp.zeros((16,), jnp.float32)
        for q in range(K // 16):
            wv[pl.ds(q * 16, 16)] = jnp.zeros((16,), jnp.float32)

        def zbody(k, _):
            z0 = pl.multiple_of(s * RPT + k * 16, 16)
            pltpu.sync_copy(rows.at[pl.ds(0, 16)],
                            accum.at[pl.ds(z0, 16)])
            pltpu.sync_copy(wv.at[pl.ds(0, 16)],
                            denacc.at[pl.ds(z0, 16)])
            return _
        lax.fori_loop(0, RPT // 16, zbody, None)

        @pl.when(s == 0)
        def _():
            pltpu.sync_copy(rows.at[pl.ds(0, 16)],
                            accum.at[pl.ds(NRANGE, 16)])
            pltpu.sync_copy(wv.at[pl.ds(0, 16)],
                            denacc.at[pl.ds(NRANGE, 16)])
        plsc.subcore_barrier()

        mslot = g * 16 + s
        moff = pl.multiple_of(16 * mslot, 16)
        mv = meta_v[pl.ds(moff, 16)]
        start = mv[0]
        nch = mv[1]

        def chunk(i, _):
            cb = pl.multiple_of(start + i * K, K)
            pltpu.sync_copy(bsrc.at[pl.ds(cb, K)], srcv)
            pltpu.sync_copy(bdst.at[pl.ds(cb, K)], dstv)
            cp = pltpu.async_copy(xp.at[srcv], rows, sem)
            cp2 = pltpu.async_copy(adt.at[dstv], adbuf, sem2)
            cp3 = pltpu.async_copy(ast.at[srcv], sbuf, sem3)
            for q in range(K // 16):
                sl = pl.ds(q * 16, 16)
                dv = dstv[sl]
                loc = dv - base
                ok = (loc >= 0) & (loc < NRANGE)
                locv[sl] = jnp.where(ok, loc, TRASH)
            cp.wait()
            cp2.wait()
            cp3.wait()
            for q in range(K // 16):
                sl = pl.ds(q * 16, 16)
                z = sbuf[sl] + adbuf[sl]
                e = jnp.where(z >= 0, z, z * jnp.float32(0.2))
                wv[sl] = jnp.exp(e)
            for jq in range(K // 16):
                wq = wv[pl.ds(jq * 16, 16)]
                for jj in range(16):
                    j = jq * 16 + jj
                    wb = jnp.full((16,), wq[jj], jnp.float32)
                    for cc in range(8):
                        sl2 = pl.ds(cc * 16, 16)
                        rows[j, sl2] = rows[j, sl2] * wb
            pltpu.sync_copy(rows, accum.at[locv], add=True)
            pltpu.sync_copy(wv, denacc.at[locv], add=True)
            return _
        lax.fori_loop(0, nch, chunk, None)
        plsc.subcore_barrier()

        def drain(k, _):
            lr0 = pl.multiple_of(s * RPT + k * 16, 16)
            gr0 = pl.multiple_of(base + lr0, 16)

            @pl.when(gr0 < N)
            def _():
                pltpu.sync_copy(accum.at[pl.ds(lr0, 16)],
                                rows.at[pl.ds(0, 16)])
                pltpu.sync_copy(denacc.at[pl.ds(lr0, 16)], dbuf)
                pltpu.sync_copy(r_h.at[pl.ds(gr0, 16)], rbuf)
                dall = dbuf[...]
                for j in range(16):
                    dn = jnp.full((16,), dall[j], jnp.float32)
                    inv = jnp.float32(1.0) / (dn + jnp.float32(1e-16))
                    for cc in range(8):
                        sl3 = pl.ds(cc * 16, 16)
                        v = rows[j, sl3] * inv + rbuf[j, sl3]
                        if do_leaky:
                            v = jnp.where(v >= 0, v, v * jnp.float32(0.01))
                        obuf[j, sl3] = v
                pltpu.sync_copy(obuf, out.at[pl.ds(gr0, 16)])
            return _
        lax.fori_loop(0, RPT // 16, drain, None)
        plsc.subcore_barrier()
        return carry

    lax.fori_loop(0, 2, one_pass, 0)


def _edge_call(do_leaky, xp, bsrc, bdst, ast, adt, r, meta):
    mesh = plsc.VectorSubcoreMesh(core_axis_name="c", subcore_axis_name="s")
    return pl.kernel(
        functools.partial(_edge_body, do_leaky),
        out_type=jax.ShapeDtypeStruct((N, D), jnp.float32),
        mesh=mesh,
        scratch_types=[
            pltpu.VMEM_SHARED((ACCROWS, D), jnp.float32),
            pltpu.VMEM_SHARED((ACCROWS,), jnp.float32),
            pltpu.VMEM((K,), jnp.int32),
            pltpu.VMEM((K,), jnp.int32),
            pltpu.VMEM((K,), jnp.int32),
            pltpu.VMEM((K,), jnp.float32),
            pltpu.VMEM((K,), jnp.float32),
            pltpu.VMEM((K,), jnp.float32),
            pltpu.VMEM((16,), jnp.float32),
            pltpu.VMEM((K, D), jnp.float32),
            pltpu.VMEM((16, D), jnp.float32),
            pltpu.VMEM((16, D), jnp.float32),
            pltpu.VMEM((1024,), jnp.int32),
            pltpu.SemaphoreType.DMA,
            pltpu.SemaphoreType.DMA,
            pltpu.SemaphoreType.DMA,
        ],
    )(xp, bsrc, bdst, ast, adt, r, meta)


def _pool_body(h3, bidx, alpha, parts, acc, hbuf, idv, alv):
    c = lax.axis_index("c")
    s = lax.axis_index("s")
    wid = c * 16 + s
    for rr in range(128 * 8):
        acc[pl.ds(rr * 16, 16)] = jnp.full((16,), jnp.float32(NEG),
                                           jnp.float32)

    def pb(k, _):
        gr = pl.multiple_of(wid * POOL_RPT + k * 16, 16)

        @pl.when(gr < N)
        def _():
            g0 = pl.multiple_of(gr * 128, 2048)
            pltpu.sync_copy(h3.at[pl.ds(g0, 16 * 128)], hbuf)
            pltpu.sync_copy(bidx.at[pl.ds(gr, 16)], idv)
            pltpu.sync_copy(alpha.at[pl.ds(gr, 16)], alv)
            idvec = idv[...]
            alvec = alv[...]
            for j in range(16):
                rid = jnp.full((16,), idvec[j] * 128, jnp.int32)
                av = jnp.full((16,), alvec[j], jnp.float32)
                for cc in range(8):
                    ixv = rid + (lax.iota(jnp.int32, 16) + cc * 16)
                    cur = plsc.load_gather(acc, [ixv])
                    val = hbuf[pl.ds(j * 128 + cc * 16, 16)] * av
                    plsc.store_scatter(acc, [ixv],
                                       jnp.maximum(cur, val))
        return _
    lax.fori_loop(0, POOL_RPT // 16, pb, None)
    pltpu.sync_copy(acc, parts.at[wid])


def _pool_call(h3, bidx, alpha):
    mesh = plsc.VectorSubcoreMesh(core_axis_name="c", subcore_axis_name="s")
    return pl.kernel(
        _pool_body,
        out_type=jax.ShapeDtypeStruct((32, 128 * 128), jnp.float32),
        mesh=mesh,
        scratch_types=[
            pltpu.VMEM((128 * 128,), jnp.float32),
            pltpu.VMEM((16 * 128,), jnp.float32),
            pltpu.VMEM((16,), jnp.int32),
            pltpu.VMEM((16,), jnp.float32),
        ],
        compiler_params=pltpu.CompilerParams(needs_layout_passes=False),
    )(h3.reshape(-1), bidx, alpha)


def _final_body(p_ref, w_ref, b_ref, o_ref):
    pv = p_ref[...]
    m = jnp.max(pv, axis=0)
    m = jnp.where(m > jnp.float32(-1e37), m, jnp.float32(0.0))
    m = jnp.where(m >= 0, m, m * jnp.float32(0.01))
    o_ref[...] = (jnp.dot(m, w_ref[...], preferred_element_type=jnp.float32)
                  + b_ref[...])


def _final_call(parts, Wout, bout):
    return pl.pallas_call(
        _final_body,
        out_shape=jax.ShapeDtypeStruct((128, 128), jnp.float32),
    )(parts, Wout, bout.reshape(1, 128))


def _bucket_edges(src, dst):
    g = (dst // NRANGE).astype(jnp.int32)
    order = jnp.argsort(g, stable=True)
    bsrc = jnp.pad(src[order], (0, 16 * K))
    bdst = jnp.pad(dst[order], (0, 16 * K), constant_values=N)
    cnt = jnp.sum(g[:, None] == jnp.arange(4, dtype=jnp.int32)[None, :],
                  axis=0).astype(jnp.int32)
    off = jnp.concatenate([jnp.zeros((1,), jnp.int32),
                           jnp.cumsum(cnt)[:3].astype(jnp.int32)])
    start = (off // K) * K
    span = off + cnt - start
    nch = (span + 16 * K - 1) // (16 * K)
    tile = jnp.arange(16, dtype=jnp.int32)
    starts = start[:, None] + tile[None, :] * nch[:, None] * K
    nchs = jnp.broadcast_to(nch[:, None], (4, 16)).astype(jnp.int32)
    pairs = jnp.concatenate(
        [starts.reshape(-1, 1).astype(jnp.int32),
         nchs.reshape(-1, 1),
         jnp.zeros((64, 14), jnp.int32)], axis=1).reshape(-1)
    return bsrc, bdst, pairs


def kernel(x, edge_index, batch_index, alpha,
           W0, as0, ad0, b0, Wr0,
           W1, as1, ad1, b1, Wr1,
           W2, as2, ad2, b2, Wr2,
           W3, as3, ad3, b3, Wr3,
           Wout, bout):
    src = edge_index[0]
    dst = edge_index[1]
    bsrc, bdst, meta = _bucket_edges(src, dst)

    h = jnp.pad(x, ((0, 0), (0, D - F_IN)))
    params = [
        (jnp.pad(W0, ((0, D - F_IN), (0, 0))), as0, ad0, b0,
         jnp.pad(Wr0, ((0, D - F_IN), (0, 0)))),
        (W1, as1, ad1, b1, Wr1),
        (W2, as2, ad2, b2, Wr2),
        (W3, as3, ad3, b3, Wr3),
    ]
    for li, (W, a_s, a_d, b, Wr) in enumerate(params):
        xp, r, av = _tc_layer(h, W, Wr, a_s, a_d, b)
        ast = jnp.pad(av[:, 0], (0, NADP - N))
        adt = jnp.pad(av[:, 1], (0, NADP - N))
        h = _edge_call(li < 3, xp, bsrc, bdst, ast, adt, r, meta)

    parts = _pool_call(h, batch_index, alpha.reshape(-1))
    return _final_call(parts.reshape(32, 128, 128), Wout, bout)


# speedup vs baseline: 21.5647x; 1.2624x over previous
import functools

import jax
import jax.numpy as jnp
from jax import lax
from jax.experimental import pallas as pl
from jax.experimental.pallas import tpu as pltpu
from jax.experimental.pallas import tpu_sc as plsc

N = 50000
D = 128
F_IN = 93
NRANGE = 12544
TRASH = NRANGE
ACCROWS = NRANGE + 16
E = 800000
K = 64
RPT = NRANGE // 16
NADP = 50176
EPAD = E + 32 * K
POOL_RPT = 1568
NEG = -3.0e38


def _mm_body(h_ref, wc_ref, a_ref, b_ref, xp_ref, r_ref, av_ref):
    hb = h_ref[...]
    y = jnp.dot(hb, wc_ref[...], preferred_element_type=jnp.float32)
    xp = y[:, :D]
    xp_ref[...] = xp
    av_ref[...] = jnp.dot(xp, a_ref[...], preferred_element_type=jnp.float32)
    r_ref[...] = y[:, D:2 * D] + b_ref[...]


def _tc_layer(h, W, Wr, att_s, att_d, b):
    wc = jnp.concatenate([W, Wr], axis=1)
    amat = jnp.stack([att_s, att_d], axis=1)
    bn = 2000
    return pl.pallas_call(
        _mm_body,
        grid=(N // bn,),
        in_specs=[pl.BlockSpec((bn, D), lambda i: (i, 0)),
                  pl.BlockSpec((D, 2 * D), lambda i: (0, 0)),
                  pl.BlockSpec((D, 2), lambda i: (0, 0)),
                  pl.BlockSpec((1, D), lambda i: (0, 0))],
        out_specs=[pl.BlockSpec((bn, D), lambda i: (i, 0)),
                   pl.BlockSpec((bn, D), lambda i: (i, 0)),
                   pl.BlockSpec((bn, 2), lambda i: (i, 0))],
        out_shape=[jax.ShapeDtypeStruct((N, D), jnp.float32),
                   jax.ShapeDtypeStruct((N, D), jnp.float32),
                   jax.ShapeDtypeStruct((N, 2), jnp.float32)],
    )(h, wc, amat, b.reshape(1, D))


def _edge_body(do_leaky, xp, bsrc, bdst, ast, adt, r_h, meta_h, out,
               accum, denacc, srcv, dstv, locv, wv, sbuf, adbuf, dbuf,
               rows, rbuf, obuf, meta_v, sem, sem2, sem3,
               srcv2, dstv2, locv2, wv2, sbuf2, adbuf2, rows2,
               semb, semb2, semb3):
    c = lax.axis_index("c")
    s = lax.axis_index("s")
    pltpu.sync_copy(meta_h, meta_v)

    def one_pass(p, carry):
        g = 2 * c + p
        base = g * NRANGE
        for j in range(16):
            for cc in range(8):
                rows[j, pl.ds(cc * 16, 16)] = jnp.zeros((16,), jnp.float32)
        for q in range(K // 16):
            wv[pl.ds(q * 16, 16)] = jnp.zeros((16,), jnp.float32)

        def zbody(k, _):
            z0 = pl.multiple_of(s * RPT + k * 16, 16)
            pltpu.sync_copy(rows.at[pl.ds(0, 16)],
                            accum.at[pl.ds(z0, 16)])
            pltpu.sync_copy(wv.at[pl.ds(0, 16)],
                            denacc.at[pl.ds(z0, 16)])
            return _
        lax.fori_loop(0, RPT // 16, zbody, None)

        @pl.when(s == 0)
        def _():
            pltpu.sync_copy(rows.at[pl.ds(0, 16)],
                            accum.at[pl.ds(NRANGE, 16)])
            pltpu.sync_copy(wv.at[pl.ds(0, 16)],
                            denacc.at[pl.ds(NRANGE, 16)])
        plsc.subcore_barrier()

        mslot = g * 16 + s
        moff = pl.multiple_of(16 * mslot, 16)
        mv = meta_v[pl.ds(moff, 16)]
        start = mv[0]
        nch = mv[1]

        def process(b_dstv, b_locv, b_wv, b_sbuf, b_adbuf, b_rows,
                    b_cp, b_cp2, b_cp3):
            for q in range(K // 16):
                sl = pl.ds(q * 16, 16)
                dv = b_dstv[sl]
                loc = dv - base
                ok = (loc >= 0) & (loc < NRANGE)
                b_locv[sl] = jnp.where(ok, loc, TRASH)
            b_cp.wait()
            b_cp2.wait()
            b_cp3.wait()
            for q in range(K // 16):
                sl = pl.ds(q * 16, 16)
                z = b_sbuf[sl] + b_adbuf[sl]
                e = jnp.where(z >= 0, z, z * jnp.float32(0.2))
                b_wv[sl] = jnp.exp(e)
            for jq in range(K // 16):
                wq = b_wv[pl.ds(jq * 16, 16)]
                for jj in range(16):
                    j = jq * 16 + jj
                    wb = jnp.full((16,), wq[jj], jnp.float32)
                    for cc in range(8):
                        sl2 = pl.ds(cc * 16, 16)
                        b_rows[j, sl2] = b_rows[j, sl2] * wb
            pltpu.sync_copy(b_rows, accum.at[b_locv], add=True)
            pltpu.sync_copy(b_wv, denacc.at[b_locv], add=True)

        def pair(i2, _):
            cb = pl.multiple_of(start + (2 * i2) * K, K)
            cb2 = pl.multiple_of(start + (2 * i2 + 1) * K, K)
            pltpu.sync_copy(bsrc.at[pl.ds(cb, K)], srcv)
            pltpu.sync_copy(bdst.at[pl.ds(cb, K)], dstv)
            cpa = pltpu.async_copy(xp.at[srcv], rows, sem)
            cpa2 = pltpu.async_copy(adt.at[dstv], adbuf, sem2)
            cpa3 = pltpu.async_copy(ast.at[srcv], sbuf, sem3)
            pltpu.sync_copy(bsrc.at[pl.ds(cb2, K)], srcv2)
            pltpu.sync_copy(bdst.at[pl.ds(cb2, K)], dstv2)
            cpb = pltpu.async_copy(xp.at[srcv2], rows2, semb)
            cpb2 = pltpu.async_copy(adt.at[dstv2], adbuf2, semb2)
            cpb3 = pltpu.async_copy(ast.at[srcv2], sbuf2, semb3)
            process(dstv, locv, wv, sbuf, adbuf, rows, cpa, cpa2, cpa3)
            process(dstv2, locv2, wv2, sbuf2, adbuf2, rows2, cpb, cpb2, cpb3)
            return _
        lax.fori_loop(0, nch // 2, pair, None)
        plsc.subcore_barrier()

        def drain(k, _):
            lr0 = pl.multiple_of(s * RPT + k * 16, 16)
            gr0 = pl.multiple_of(base + lr0, 16)

            @pl.when(gr0 < N)
            def _():
                pltpu.sync_copy(accum.at[pl.ds(lr0, 16)],
                                rows.at[pl.ds(0, 16)])
                pltpu.sync_copy(denacc.at[pl.ds(lr0, 16)], dbuf)
                pltpu.sync_copy(r_h.at[pl.ds(gr0, 16)], rbuf)
                dall = dbuf[...]
                for j in range(16):
                    dn = jnp.full((16,), dall[j], jnp.float32)
                    inv = jnp.float32(1.0) / (dn + jnp.float32(1e-16))
                    for cc in range(8):
                        sl3 = pl.ds(cc * 16, 16)
                        v = rows[j, sl3] * inv + rbuf[j, sl3]
                        if do_leaky:
                            v = jnp.where(v >= 0, v, v * jnp.float32(0.01))
                        obuf[j, sl3] = v
                pltpu.sync_copy(obuf, out.at[pl.ds(gr0, 16)])
            return _
        lax.fori_loop(0, RPT // 16, drain, None)
        plsc.subcore_barrier()
        return carry

    lax.fori_loop(0, 2, one_pass, 0)


def _edge_call(do_leaky, xp, bsrc, bdst, ast, adt, r, meta):
    mesh = plsc.VectorSubcoreMesh(core_axis_name="c", subcore_axis_name="s")
    return pl.kernel(
        functools.partial(_edge_body, do_leaky),
        out_type=jax.ShapeDtypeStruct((N, D), jnp.float32),
        mesh=mesh,
        scratch_types=[
            pltpu.VMEM_SHARED((ACCROWS, D), jnp.float32),
            pltpu.VMEM_SHARED((ACCROWS,), jnp.float32),
            pltpu.VMEM((K,), jnp.int32),
            pltpu.VMEM((K,), jnp.int32),
            pltpu.VMEM((K,), jnp.int32),
            pltpu.VMEM((K,), jnp.float32),
            pltpu.VMEM((K,), jnp.float32),
            pltpu.VMEM((K,), jnp.float32),
            pltpu.VMEM((16,), jnp.float32),
            pltpu.VMEM((K, D), jnp.float32),
            pltpu.VMEM((16, D), jnp.float32),
            pltpu.VMEM((16, D), jnp.float32),
            pltpu.VMEM((1024,), jnp.int32),
            pltpu.SemaphoreType.DMA,
            pltpu.SemaphoreType.DMA,
            pltpu.SemaphoreType.DMA,
            pltpu.VMEM((K,), jnp.int32),
            pltpu.VMEM((K,), jnp.int32),
            pltpu.VMEM((K,), jnp.int32),
            pltpu.VMEM((K,), jnp.float32),
            pltpu.VMEM((K,), jnp.float32),
            pltpu.VMEM((K,), jnp.float32),
            pltpu.VMEM((K, D), jnp.float32),
            pltpu.SemaphoreType.DMA,
            pltpu.SemaphoreType.DMA,
            pltpu.SemaphoreType.DMA,
        ],
    )(xp, bsrc, bdst, ast, adt, r, meta)


def _pool_body(h3, bidx, alpha, parts, acc, hbuf, idv, alv):
    c = lax.axis_index("c")
    s = lax.axis_index("s")
    wid = c * 16 + s
    for rr in range(128 * 8):
        acc[pl.ds(rr * 16, 16)] = jnp.full((16,), jnp.float32(NEG),
                                           jnp.float32)

    def pb(k, _):
        gr = pl.multiple_of(wid * POOL_RPT + k * 16, 16)

        @pl.when(gr < N)
        def _():
            g0 = pl.multiple_of(gr * 128, 2048)
            pltpu.sync_copy(h3.at[pl.ds(g0, 16 * 128)], hbuf)
            pltpu.sync_copy(bidx.at[pl.ds(gr, 16)], idv)
            pltpu.sync_copy(alpha.at[pl.ds(gr, 16)], alv)
            idvec = idv[...]
            alvec = alv[...]
            for j in range(16):
                rid = jnp.full((16,), idvec[j] * 128, jnp.int32)
                av = jnp.full((16,), alvec[j], jnp.float32)
                for cc in range(8):
                    ixv = rid + (lax.iota(jnp.int32, 16) + cc * 16)
                    cur = plsc.load_gather(acc, [ixv])
                    val = hbuf[pl.ds(j * 128 + cc * 16, 16)] * av
                    plsc.store_scatter(acc, [ixv],
                                       jnp.maximum(cur, val))
        return _
    lax.fori_loop(0, POOL_RPT // 16, pb, None)
    pltpu.sync_copy(acc, parts.at[wid])


def _pool_call(h3, bidx, alpha):
    mesh = plsc.VectorSubcoreMesh(core_axis_name="c", subcore_axis_name="s")
    return pl.kernel(
        _pool_body,
        out_type=jax.ShapeDtypeStruct((32, 128 * 128), jnp.float32),
        mesh=mesh,
        scratch_types=[
            pltpu.VMEM((128 * 128,), jnp.float32),
            pltpu.VMEM((16 * 128,), jnp.float32),
            pltpu.VMEM((16,), jnp.int32),
            pltpu.VMEM((16,), jnp.float32),
        ],
        compiler_params=pltpu.CompilerParams(needs_layout_passes=False),
    )(h3.reshape(-1), bidx, alpha)


def _final_body(p_ref, w_ref, b_ref, o_ref):
    pv = p_ref[...]
    m = jnp.max(pv, axis=0)
    m = jnp.where(m > jnp.float32(-1e37), m, jnp.float32(0.0))
    m = jnp.where(m >= 0, m, m * jnp.float32(0.01))
    o_ref[...] = (jnp.dot(m, w_ref[...], preferred_element_type=jnp.float32)
                  + b_ref[...])


def _final_call(parts, Wout, bout):
    return pl.pallas_call(
        _final_body,
        out_shape=jax.ShapeDtypeStruct((128, 128), jnp.float32),
    )(parts, Wout, bout.reshape(1, 128))


def _bucket_edges(src, dst):
    g = (dst // NRANGE).astype(jnp.int32)
    order = jnp.argsort(g, stable=True)
    bsrc = jnp.pad(src[order], (0, 32 * K))
    bdst = jnp.pad(dst[order], (0, 32 * K), constant_values=N)
    cnt = jnp.sum(g[:, None] == jnp.arange(4, dtype=jnp.int32)[None, :],
                  axis=0).astype(jnp.int32)
    off = jnp.concatenate([jnp.zeros((1,), jnp.int32),
                           jnp.cumsum(cnt)[:3].astype(jnp.int32)])
    start = (off // K) * K
    span = off + cnt - start
    nch = 2 * ((span + 32 * K - 1) // (32 * K))
    tile = jnp.arange(16, dtype=jnp.int32)
    starts = start[:, None] + tile[None, :] * nch[:, None] * K
    nchs = jnp.broadcast_to(nch[:, None], (4, 16)).astype(jnp.int32)
    pairs = jnp.concatenate(
        [starts.reshape(-1, 1).astype(jnp.int32),
         nchs.reshape(-1, 1),
         jnp.zeros((64, 14), jnp.int32)], axis=1).reshape(-1)
    return bsrc, bdst, pairs


def kernel(x, edge_index, batch_index, alpha,
           W0, as0, ad0, b0, Wr0,
           W1, as1, ad1, b1, Wr1,
           W2, as2, ad2, b2, Wr2,
           W3, as3, ad3, b3, Wr3,
           Wout, bout):
    src = edge_index[0]
    dst = edge_index[1]
    bsrc, bdst, meta = _bucket_edges(src, dst)

    h = jnp.pad(x, ((0, 0), (0, D - F_IN)))
    params = [
        (jnp.pad(W0, ((0, D - F_IN), (0, 0))), as0, ad0, b0,
         jnp.pad(Wr0, ((0, D - F_IN), (0, 0)))),
        (W1, as1, ad1, b1, Wr1),
        (W2, as2, ad2, b2, Wr2),
        (W3, as3, ad3, b3, Wr3),
    ]
    for li, (W, a_s, a_d, b, Wr) in enumerate(params):
        xp, r, av = _tc_layer(h, W, Wr, a_s, a_d, b)
        ast = jnp.pad(av[:, 0], (0, NADP - N))
        adt = jnp.pad(av[:, 1], (0, NADP - N))
        h = _edge_call(li < 3, xp, bsrc, bdst, ast, adt, r, meta)

    parts = _pool_call(h, batch_index, alpha.reshape(-1))
    return _final_call(parts.reshape(32, 128, 128), Wout, bout)


# speedup vs baseline: 22.4532x; 1.0412x over previous
import functools

import jax
import jax.numpy as jnp
from jax import lax
from jax.experimental import pallas as pl
from jax.experimental.pallas import tpu as pltpu
from jax.experimental.pallas import tpu_sc as plsc

N = 50000
D = 128
F_IN = 93
NRANGE = 12544
TRASH = NRANGE
ACCROWS = NRANGE + 16
E = 800000
K = 64
RPT = NRANGE // 16
NADP = 50176
EPAD = E + 32 * K
POOL_RPT = 1568
NEG = -3.0e38


def _mm_body(h_ref, wc_ref, a_ref, b_ref, xp_ref, r_ref, av_ref):
    hb = h_ref[...]
    y = jnp.dot(hb, wc_ref[...], preferred_element_type=jnp.float32)
    xp = y[:, :D]
    xp_ref[...] = xp
    av_ref[...] = jnp.dot(xp, a_ref[...], preferred_element_type=jnp.float32)
    r_ref[...] = y[:, D:2 * D] + b_ref[...]


def _tc_layer(h, W, Wr, att_s, att_d, b):
    wc = jnp.concatenate([W, Wr], axis=1)
    amat = jnp.stack([att_s, att_d], axis=1)
    bn = 2000
    return pl.pallas_call(
        _mm_body,
        grid=(N // bn,),
        in_specs=[pl.BlockSpec((bn, D), lambda i: (i, 0)),
                  pl.BlockSpec((D, 2 * D), lambda i: (0, 0)),
                  pl.BlockSpec((D, 2), lambda i: (0, 0)),
                  pl.BlockSpec((1, D), lambda i: (0, 0))],
        out_specs=[pl.BlockSpec((bn, D), lambda i: (i, 0)),
                   pl.BlockSpec((bn, D), lambda i: (i, 0)),
                   pl.BlockSpec((bn, 2), lambda i: (i, 0))],
        out_shape=[jax.ShapeDtypeStruct((N, D), jnp.float32),
                   jax.ShapeDtypeStruct((N, D), jnp.float32),
                   jax.ShapeDtypeStruct((N, 2), jnp.float32)],
    )(h, wc, amat, b.reshape(1, D))


def _edge_body(do_leaky, xp, bsrc, bdst, ast, adt, r_h, meta_h, out,
               accum, denacc, srcv, dstv, locv, wv, sbuf, adbuf, dbuf,
               rows, rbuf, obuf, meta_v, sem, sem2, sem3,
               srcv2, dstv2, locv2, wv2, sbuf2, adbuf2, rows2,
               semb, semb2, semb3):
    c = lax.axis_index("c")
    s = lax.axis_index("s")
    pltpu.sync_copy(meta_h, meta_v)

    def one_pass(p, carry):
        g = 2 * c + p
        base = g * NRANGE
        for j in range(K):
            for cc in range(8):
                rows[j, pl.ds(cc * 16, 16)] = jnp.zeros((16,), jnp.float32)
        for q in range(K // 16):
            wv[pl.ds(q * 16, 16)] = jnp.zeros((16,), jnp.float32)

        zcps = []
        for k in range(RPT // K):
            z0 = pl.multiple_of(s * RPT + k * K, 16)
            zcps.append(pltpu.async_copy(rows, accum.at[pl.ds(z0, K)], sem))
            zcps.append(pltpu.async_copy(wv, denacc.at[pl.ds(z0, K)], sem2))
        zt = pl.multiple_of(s * RPT + (RPT // K) * K, 16)
        zcps.append(pltpu.async_copy(rows.at[pl.ds(0, RPT % K)],
                                     accum.at[pl.ds(zt, RPT % K)], sem))
        zcps.append(pltpu.async_copy(wv.at[pl.ds(0, RPT % K)],
                                     denacc.at[pl.ds(zt, RPT % K)], sem2))

        @pl.when(s == 0)
        def _():
            pltpu.sync_copy(rows.at[pl.ds(0, 16)],
                            accum.at[pl.ds(NRANGE, 16)])
            pltpu.sync_copy(wv.at[pl.ds(0, 16)],
                            denacc.at[pl.ds(NRANGE, 16)])
        for zcp in zcps:
            zcp.wait()
        plsc.subcore_barrier()

        mslot = g * 16 + s
        moff = pl.multiple_of(16 * mslot, 16)
        mv = meta_v[pl.ds(moff, 16)]
        start = mv[0]
        nch = mv[1]

        def process(b_dstv, b_locv, b_wv, b_sbuf, b_adbuf, b_rows,
                    b_cp, b_cp2, b_cp3):
            for q in range(K // 16):
                sl = pl.ds(q * 16, 16)
                dv = b_dstv[sl]
                loc = dv - base
                ok = (loc >= 0) & (loc < NRANGE)
                b_locv[sl] = jnp.where(ok, loc, TRASH)
            b_cp.wait()
            b_cp2.wait()
            b_cp3.wait()
            for q in range(K // 16):
                sl = pl.ds(q * 16, 16)
                z = b_sbuf[sl] + b_adbuf[sl]
                e = jnp.where(z >= 0, z, z * jnp.float32(0.2))
                b_wv[sl] = jnp.exp(e)
            for jq in range(K // 16):
                wq = b_wv[pl.ds(jq * 16, 16)]
                for jj in range(16):
                    j = jq * 16 + jj
                    wb = jnp.full((16,), wq[jj], jnp.float32)
                    for cc in range(8):
                        sl2 = pl.ds(cc * 16, 16)
                        b_rows[j, sl2] = b_rows[j, sl2] * wb
            pltpu.sync_copy(b_rows, accum.at[b_locv], add=True)
            pltpu.sync_copy(b_wv, denacc.at[b_locv], add=True)

        def pair(i2, _):
            cb = pl.multiple_of(start + (2 * i2) * K, K)
            cb2 = pl.multiple_of(start + (2 * i2 + 1) * K, K)
            pltpu.sync_copy(bsrc.at[pl.ds(cb, K)], srcv)
            pltpu.sync_copy(bdst.at[pl.ds(cb, K)], dstv)
            cpa = pltpu.async_copy(xp.at[srcv], rows, sem)
            cpa2 = pltpu.async_copy(adt.at[dstv], adbuf, sem2)
            cpa3 = pltpu.async_copy(ast.at[srcv], sbuf, sem3)
            pltpu.sync_copy(bsrc.at[pl.ds(cb2, K)], srcv2)
            pltpu.sync_copy(bdst.at[pl.ds(cb2, K)], dstv2)
            cpb = pltpu.async_copy(xp.at[srcv2], rows2, semb)
            cpb2 = pltpu.async_copy(adt.at[dstv2], adbuf2, semb2)
            cpb3 = pltpu.async_copy(ast.at[srcv2], sbuf2, semb3)
            process(dstv, locv, wv, sbuf, adbuf, rows, cpa, cpa2, cpa3)
            process(dstv2, locv2, wv2, sbuf2, adbuf2, rows2, cpb, cpb2, cpb3)
            return _
        lax.fori_loop(0, nch // 2, pair, None)
        plsc.subcore_barrier()

        def drain(k, _):
            lr0 = pl.multiple_of(s * RPT + k * 16, 16)
            gr0 = pl.multiple_of(base + lr0, 16)

            @pl.when(gr0 < N)
            def _():
                pltpu.sync_copy(accum.at[pl.ds(lr0, 16)],
                                rows.at[pl.ds(0, 16)])
                pltpu.sync_copy(denacc.at[pl.ds(lr0, 16)], dbuf)
                pltpu.sync_copy(r_h.at[pl.ds(gr0, 16)], rbuf)
                dall = dbuf[...]
                for j in range(16):
                    dn = jnp.full((16,), dall[j], jnp.float32)
                    inv = jnp.float32(1.0) / (dn + jnp.float32(1e-16))
                    for cc in range(8):
                        sl3 = pl.ds(cc * 16, 16)
                        v = rows[j, sl3] * inv + rbuf[j, sl3]
                        if do_leaky:
                            v = jnp.where(v >= 0, v, v * jnp.float32(0.01))
                        obuf[j, sl3] = v
                pltpu.sync_copy(obuf, out.at[pl.ds(gr0, 16)])
            return _
        lax.fori_loop(0, RPT // 16, drain, None)
        plsc.subcore_barrier()
        return carry

    lax.fori_loop(0, 2, one_pass, 0)


def _edge_call(do_leaky, xp, bsrc, bdst, ast, adt, r, meta):
    mesh = plsc.VectorSubcoreMesh(core_axis_name="c", subcore_axis_name="s")
    return pl.kernel(
        functools.partial(_edge_body, do_leaky),
        out_type=jax.ShapeDtypeStruct((N, D), jnp.float32),
        mesh=mesh,
        scratch_types=[
            pltpu.VMEM_SHARED((ACCROWS, D), jnp.float32),
            pltpu.VMEM_SHARED((ACCROWS,), jnp.float32),
            pltpu.VMEM((K,), jnp.int32),
            pltpu.VMEM((K,), jnp.int32),
            pltpu.VMEM((K,), jnp.int32),
            pltpu.VMEM((K,), jnp.float32),
            pltpu.VMEM((K,), jnp.float32),
            pltpu.VMEM((K,), jnp.float32),
            pltpu.VMEM((16,), jnp.float32),
            pltpu.VMEM((K, D), jnp.float32),
            pltpu.VMEM((16, D), jnp.float32),
            pltpu.VMEM((16, D), jnp.float32),
            pltpu.VMEM((1024,), jnp.int32),
            pltpu.SemaphoreType.DMA,
            pltpu.SemaphoreType.DMA,
            pltpu.SemaphoreType.DMA,
            pltpu.VMEM((K,), jnp.int32),
            pltpu.VMEM((K,), jnp.int32),
            pltpu.VMEM((K,), jnp.int32),
            pltpu.VMEM((K,), jnp.float32),
            pltpu.VMEM((K,), jnp.float32),
            pltpu.VMEM((K,), jnp.float32),
            pltpu.VMEM((K, D), jnp.float32),
            pltpu.SemaphoreType.DMA,
            pltpu.SemaphoreType.DMA,
            pltpu.SemaphoreType.DMA,
        ],
    )(xp, bsrc, bdst, ast, adt, r, meta)


def _pool_body(h3, bidx, alpha, parts, acc, hbuf, idv, alv):
    c = lax.axis_index("c")
    s = lax.axis_index("s")
    wid = c * 16 + s
    for rr in range(128 * 8):
        acc[pl.ds(rr * 16, 16)] = jnp.full((16,), jnp.float32(NEG),
                                           jnp.float32)

    def pb(k, _):
        gr = pl.multiple_of(wid * POOL_RPT + k * 16, 16)

        @pl.when(gr < N)
        def _():
            g0 = pl.multiple_of(gr * 128, 2048)
            pltpu.sync_copy(h3.at[pl.ds(g0, 16 * 128)], hbuf)
            pltpu.sync_copy(bidx.at[pl.ds(gr, 16)], idv)
            pltpu.sync_copy(alpha.at[pl.ds(gr, 16)], alv)
            idvec = idv[...]
            alvec = alv[...]
            for j in range(16):
                rid = jnp.full((16,), idvec[j] * 128, jnp.int32)
                av = jnp.full((16,), alvec[j], jnp.float32)
                for cc in range(8):
                    ixv = rid + (lax.iota(jnp.int32, 16) + cc * 16)
                    cur = plsc.load_gather(acc, [ixv])
                    val = hbuf[pl.ds(j * 128 + cc * 16, 16)] * av
                    plsc.store_scatter(acc, [ixv],
                                       jnp.maximum(cur, val))
        return _
    lax.fori_loop(0, POOL_RPT // 16, pb, None)
    pltpu.sync_copy(acc, parts.at[wid])


def _pool_call(h3, bidx, alpha):
    mesh = plsc.VectorSubcoreMesh(core_axis_name="c", subcore_axis_name="s")
    return pl.kernel(
        _pool_body,
        out_type=jax.ShapeDtypeStruct((32, 128 * 128), jnp.float32),
        mesh=mesh,
        scratch_types=[
            pltpu.VMEM((128 * 128,), jnp.float32),
            pltpu.VMEM((16 * 128,), jnp.float32),
            pltpu.VMEM((16,), jnp.int32),
            pltpu.VMEM((16,), jnp.float32),
        ],
        compiler_params=pltpu.CompilerParams(needs_layout_passes=False),
    )(h3.reshape(-1), bidx, alpha)


def _final_body(p_ref, w_ref, b_ref, o_ref):
    pv = p_ref[...]
    m = jnp.max(pv, axis=0)
    m = jnp.where(m > jnp.float32(-1e37), m, jnp.float32(0.0))
    m = jnp.where(m >= 0, m, m * jnp.float32(0.01))
    o_ref[...] = (jnp.dot(m, w_ref[...], preferred_element_type=jnp.float32)
                  + b_ref[...])


def _final_call(parts, Wout, bout):
    return pl.pallas_call(
        _final_body,
        out_shape=jax.ShapeDtypeStruct((128, 128), jnp.float32),
    )(parts, Wout, bout.reshape(1, 128))


def _bucket_edges(src, dst):
    g = (dst // NRANGE).astype(jnp.int32)
    gs, src_s, dst_s = lax.sort((g, src, dst), num_keys=1, is_stable=False)
    bsrc = jnp.pad(src_s, (0, 32 * K))
    bdst = jnp.pad(dst_s, (0, 32 * K), constant_values=N)
    off = jnp.searchsorted(gs, jnp.arange(4, dtype=jnp.int32),
                           side="left").astype(jnp.int32)
    cnt = (jnp.concatenate([off[1:], jnp.full((1,), E, jnp.int32)])
           - off).astype(jnp.int32)
    start = (off // K) * K
    span = off + cnt - start
    nch = 2 * ((span + 32 * K - 1) // (32 * K))
    tile = jnp.arange(16, dtype=jnp.int32)
    starts = start[:, None] + tile[None, :] * nch[:, None] * K
    nchs = jnp.broadcast_to(nch[:, None], (4, 16)).astype(jnp.int32)
    pairs = jnp.concatenate(
        [starts.reshape(-1, 1).astype(jnp.int32),
         nchs.reshape(-1, 1),
         jnp.zeros((64, 14), jnp.int32)], axis=1).reshape(-1)
    return bsrc, bdst, pairs


def kernel(x, edge_index, batch_index, alpha,
           W0, as0, ad0, b0, Wr0,
           W1, as1, ad1, b1, Wr1,
           W2, as2, ad2, b2, Wr2,
           W3, as3, ad3, b3, Wr3,
           Wout, bout):
    src = edge_index[0]
    dst = edge_index[1]
    bsrc, bdst, meta = _bucket_edges(src, dst)

    h = jnp.pad(x, ((0, 0), (0, D - F_IN)))
    params = [
        (jnp.pad(W0, ((0, D - F_IN), (0, 0))), as0, ad0, b0,
         jnp.pad(Wr0, ((0, D - F_IN), (0, 0)))),
        (W1, as1, ad1, b1, Wr1),
        (W2, as2, ad2, b2, Wr2),
        (W3, as3, ad3, b3, Wr3),
    ]
    for li, (W, a_s, a_d, b, Wr) in enumerate(params):
        xp, r, av = _tc_layer(h, W, Wr, a_s, a_d, b)
        ast = jnp.pad(av[:, 0], (0, NADP - N))
        adt = jnp.pad(av[:, 1], (0, NADP - N))
        h = _edge_call(li < 3, xp, bsrc, bdst, ast, adt, r, meta)

    parts = _pool_call(h, batch_index, alpha.reshape(-1))
    return _final_call(parts.reshape(32, 128, 128), Wout, bout)


# speedup vs baseline: 23.5038x; 1.0468x over previous
import functools

import jax
import jax.numpy as jnp
from jax import lax
from jax.experimental import pallas as pl
from jax.experimental.pallas import tpu as pltpu
from jax.experimental.pallas import tpu_sc as plsc

N = 50000
D = 128
F_IN = 93
NRANGE = 12544
TRASH = NRANGE
ACCROWS = NRANGE + 16
E = 800000
K = 64
RPT = NRANGE // 16
NADP = 50176
EPAD = E + 32 * K
POOL_RPT = 1568
NEG = -3.0e38


def _mm_body(h_ref, wc_ref, a_ref, b_ref, xp_ref, r_ref, av_ref):
    hb = h_ref[...]
    y = jnp.dot(hb, wc_ref[...], preferred_element_type=jnp.float32)
    xp = y[:, :D]
    xp_ref[...] = xp
    av_ref[...] = jnp.dot(xp, a_ref[...], preferred_element_type=jnp.float32)
    r_ref[...] = y[:, D:2 * D] + b_ref[...]


def _tc_layer(h, W, Wr, att_s, att_d, b):
    wc = jnp.concatenate([W, Wr], axis=1)
    amat = jnp.stack([att_s, att_d], axis=1)
    bn = 2000
    return pl.pallas_call(
        _mm_body,
        grid=(N // bn,),
        in_specs=[pl.BlockSpec((bn, D), lambda i: (i, 0)),
                  pl.BlockSpec((D, 2 * D), lambda i: (0, 0)),
                  pl.BlockSpec((D, 2), lambda i: (0, 0)),
                  pl.BlockSpec((1, D), lambda i: (0, 0))],
        out_specs=[pl.BlockSpec((bn, D), lambda i: (i, 0)),
                   pl.BlockSpec((bn, D), lambda i: (i, 0)),
                   pl.BlockSpec((bn, 2), lambda i: (i, 0))],
        out_shape=[jax.ShapeDtypeStruct((N, D), jnp.float32),
                   jax.ShapeDtypeStruct((N, D), jnp.float32),
                   jax.ShapeDtypeStruct((N, 2), jnp.float32)],
    )(h, wc, amat, b.reshape(1, D))


def _edge_body(do_leaky, xp, bsrc, bdst, ast, adt, r_h, meta_h, out,
               accum, denacc, srcv, dstv, locv, wv, sbuf, adbuf, dbuf,
               rows, rbuf, obuf, meta_v, sem, sem2, sem3,
               srcv2, dstv2, locv2, wv2, sbuf2, adbuf2, rows2,
               semb, semb2, semb3, sems1, semw1, sems2, semw2):
    c = lax.axis_index("c")
    s = lax.axis_index("s")
    pltpu.sync_copy(meta_h, meta_v)

    def one_pass(p, carry):
        g = 2 * c + p
        base = g * NRANGE
        for j in range(K):
            for cc in range(8):
                rows[j, pl.ds(cc * 16, 16)] = jnp.zeros((16,), jnp.float32)
        for q in range(K // 16):
            wv[pl.ds(q * 16, 16)] = jnp.zeros((16,), jnp.float32)

        zcps = []
        for k in range(RPT // K):
            z0 = pl.multiple_of(s * RPT + k * K, 16)
            zcps.append(pltpu.async_copy(rows, accum.at[pl.ds(z0, K)], sem))
            zcps.append(pltpu.async_copy(wv, denacc.at[pl.ds(z0, K)], sem2))
        zt = pl.multiple_of(s * RPT + (RPT // K) * K, 16)
        zcps.append(pltpu.async_copy(rows.at[pl.ds(0, RPT % K)],
                                     accum.at[pl.ds(zt, RPT % K)], sem))
        zcps.append(pltpu.async_copy(wv.at[pl.ds(0, RPT % K)],
                                     denacc.at[pl.ds(zt, RPT % K)], sem2))

        @pl.when(s == 0)
        def _():
            pltpu.sync_copy(rows.at[pl.ds(0, 16)],
                            accum.at[pl.ds(NRANGE, 16)])
            pltpu.sync_copy(wv.at[pl.ds(0, 16)],
                            denacc.at[pl.ds(NRANGE, 16)])
        for zcp in zcps:
            zcp.wait()
        plsc.subcore_barrier()

        mslot = g * 16 + s
        moff = pl.multiple_of(16 * mslot, 16)
        mv = meta_v[pl.ds(moff, 16)]
        start = mv[0]
        nch = mv[1]

        def process(b_dstv, b_locv, b_wv, b_sbuf, b_adbuf, b_rows,
                    b_cp, b_cp2, b_cp3, b_sems, b_semw):
            for q in range(K // 16):
                sl = pl.ds(q * 16, 16)
                dv = b_dstv[sl]
                loc = dv - base
                ok = (loc >= 0) & (loc < NRANGE)
                b_locv[sl] = jnp.where(ok, loc, TRASH)
            b_cp.wait()
            b_cp2.wait()
            b_cp3.wait()
            for q in range(K // 16):
                sl = pl.ds(q * 16, 16)
                z = b_sbuf[sl] + b_adbuf[sl]
                e = jnp.where(z >= 0, z, z * jnp.float32(0.2))
                b_wv[sl] = jnp.exp(e)
            for jq in range(K // 16):
                wq = b_wv[pl.ds(jq * 16, 16)]
                for jj in range(16):
                    j = jq * 16 + jj
                    wb = jnp.full((16,), wq[jj], jnp.float32)
                    for cc in range(8):
                        sl2 = pl.ds(cc * 16, 16)
                        b_rows[j, sl2] = b_rows[j, sl2] * wb
            pltpu.async_copy(b_rows, accum.at[b_locv], b_sems, add=True)
            pltpu.async_copy(b_wv, denacc.at[b_locv], b_semw, add=True)

        def drain_scatters():
            pltpu.make_async_copy(xp.at[pl.ds(0, K)], rows, sems1).wait()
            pltpu.make_async_copy(ast.at[pl.ds(0, K)], wv, semw1).wait()
            pltpu.make_async_copy(xp.at[pl.ds(0, K)], rows2, sems2).wait()
            pltpu.make_async_copy(ast.at[pl.ds(0, K)], wv2, semw2).wait()

        def pair(i2, _):
            @pl.when(i2 > 0)
            def _():
                drain_scatters()
            cb = pl.multiple_of(start + (2 * i2) * K, K)
            cb2 = pl.multiple_of(start + (2 * i2 + 1) * K, K)
            pltpu.sync_copy(bsrc.at[pl.ds(cb, K)], srcv)
            pltpu.sync_copy(bdst.at[pl.ds(cb, K)], dstv)
            cpa = pltpu.async_copy(xp.at[srcv], rows, sem)
            cpa2 = pltpu.async_copy(adt.at[dstv], adbuf, sem2)
            cpa3 = pltpu.async_copy(ast.at[srcv], sbuf, sem3)
            pltpu.sync_copy(bsrc.at[pl.ds(cb2, K)], srcv2)
            pltpu.sync_copy(bdst.at[pl.ds(cb2, K)], dstv2)
            cpb = pltpu.async_copy(xp.at[srcv2], rows2, semb)
            cpb2 = pltpu.async_copy(adt.at[dstv2], adbuf2, semb2)
            cpb3 = pltpu.async_copy(ast.at[srcv2], sbuf2, semb3)
            process(dstv, locv, wv, sbuf, adbuf, rows, cpa, cpa2, cpa3,
                    sems1, semw1)
            process(dstv2, locv2, wv2, sbuf2, adbuf2, rows2, cpb, cpb2, cpb3,
                    sems2, semw2)
            return _
        lax.fori_loop(0, nch // 2, pair, None)

        @pl.when(nch > 0)
        def _():
            drain_scatters()
        plsc.subcore_barrier()

        def drain(k, _):
            lr0 = pl.multiple_of(s * RPT + k * 16, 16)
            gr0 = pl.multiple_of(base + lr0, 16)

            @pl.when(gr0 < N)
            def _():
                ca = pltpu.async_copy(accum.at[pl.ds(lr0, 16)],
                                      rows.at[pl.ds(0, 16)], sem)
                cd = pltpu.async_copy(denacc.at[pl.ds(lr0, 16)], dbuf, sem2)
                cr = pltpu.async_copy(r_h.at[pl.ds(gr0, 16)], rbuf, sem3)

                @pl.when(k > 0)
                def _():
                    pltpu.make_async_copy(r_h.at[pl.ds(gr0, 16)], obuf,
                                          semb).wait()
                ca.wait()
                cd.wait()
                cr.wait()
                dall = dbuf[...]
                for j in range(16):
                    dn = jnp.full((16,), dall[j], jnp.float32)
                    inv = jnp.float32(1.0) / (dn + jnp.float32(1e-16))
                    for cc in range(8):
                        sl3 = pl.ds(cc * 16, 16)
                        v = rows[j, sl3] * inv + rbuf[j, sl3]
                        if do_leaky:
                            v = jnp.where(v >= 0, v, v * jnp.float32(0.01))
                        obuf[j, sl3] = v
                pltpu.async_copy(obuf, out.at[pl.ds(gr0, 16)], semb)
            return _
        lax.fori_loop(0, RPT // 16, drain, None)

        @pl.when(base + s * RPT < N)
        def _():
            pltpu.make_async_copy(r_h.at[pl.ds(0, 16)], obuf, semb).wait()
        plsc.subcore_barrier()
        return carry

    lax.fori_loop(0, 2, one_pass, 0)


def _edge_call(do_leaky, xp, bsrc, bdst, ast, adt, r, meta):
    mesh = plsc.VectorSubcoreMesh(core_axis_name="c", subcore_axis_name="s")
    return pl.kernel(
        functools.partial(_edge_body, do_leaky),
        out_type=jax.ShapeDtypeStruct((N, D), jnp.float32),
        mesh=mesh,
        scratch_types=[
            pltpu.VMEM_SHARED((ACCROWS, D), jnp.float32),
            pltpu.VMEM_SHARED((ACCROWS,), jnp.float32),
            pltpu.VMEM((K,), jnp.int32),
            pltpu.VMEM((K,), jnp.int32),
            pltpu.VMEM((K,), jnp.int32),
            pltpu.VMEM((K,), jnp.float32),
            pltpu.VMEM((K,), jnp.float32),
            pltpu.VMEM((K,), jnp.float32),
            pltpu.VMEM((16,), jnp.float32),
            pltpu.VMEM((K, D), jnp.float32),
            pltpu.VMEM((16, D), jnp.float32),
            pltpu.VMEM((16, D), jnp.float32),
            pltpu.VMEM((1024,), jnp.int32),
            pltpu.SemaphoreType.DMA,
            pltpu.SemaphoreType.DMA,
            pltpu.SemaphoreType.DMA,
            pltpu.VMEM((K,), jnp.int32),
            pltpu.VMEM((K,), jnp.int32),
            pltpu.VMEM((K,), jnp.int32),
            pltpu.VMEM((K,), jnp.float32),
            pltpu.VMEM((K,), jnp.float32),
            pltpu.VMEM((K,), jnp.float32),
            pltpu.VMEM((K, D), jnp.float32),
            pltpu.SemaphoreType.DMA,
            pltpu.SemaphoreType.DMA,
            pltpu.SemaphoreType.DMA,
            pltpu.SemaphoreType.DMA,
            pltpu.SemaphoreType.DMA,
            pltpu.SemaphoreType.DMA,
            pltpu.SemaphoreType.DMA,
        ],
    )(xp, bsrc, bdst, ast, adt, r, meta)


def _pool_body(h3, bidx, alpha, parts, acc, hbuf, idv, alv):
    c = lax.axis_index("c")
    s = lax.axis_index("s")
    wid = c * 16 + s
    for rr in range(128 * 8):
        acc[pl.ds(rr * 16, 16)] = jnp.full((16,), jnp.float32(NEG),
                                           jnp.float32)

    def pb(k, _):
        gr = pl.multiple_of(wid * POOL_RPT + k * 16, 16)

        @pl.when(gr < N)
        def _():
            g0 = pl.multiple_of(gr * 128, 2048)
            pltpu.sync_copy(h3.at[pl.ds(g0, 16 * 128)], hbuf)
            pltpu.sync_copy(bidx.at[pl.ds(gr, 16)], idv)
            pltpu.sync_copy(alpha.at[pl.ds(gr, 16)], alv)
            idvec = idv[...]
            alvec = alv[...]
            for j in range(16):
                rid = jnp.full((16,), idvec[j] * 128, jnp.int32)
                av = jnp.full((16,), alvec[j], jnp.float32)
                for cc in range(8):
                    ixv = rid + (lax.iota(jnp.int32, 16) + cc * 16)
                    cur = plsc.load_gather(acc, [ixv])
                    val = hbuf[pl.ds(j * 128 + cc * 16, 16)] * av
                    plsc.store_scatter(acc, [ixv],
                                       jnp.maximum(cur, val))
        return _
    lax.fori_loop(0, POOL_RPT // 16, pb, None)
    pltpu.sync_copy(acc, parts.at[wid])


def _pool_call(h3, bidx, alpha):
    mesh = plsc.VectorSubcoreMesh(core_axis_name="c", subcore_axis_name="s")
    return pl.kernel(
        _pool_body,
        out_type=jax.ShapeDtypeStruct((32, 128 * 128), jnp.float32),
        mesh=mesh,
        scratch_types=[
            pltpu.VMEM((128 * 128,), jnp.float32),
            pltpu.VMEM((16 * 128,), jnp.float32),
            pltpu.VMEM((16,), jnp.int32),
            pltpu.VMEM((16,), jnp.float32),
        ],
        compiler_params=pltpu.CompilerParams(needs_layout_passes=False),
    )(h3.reshape(-1), bidx, alpha)


def _final_body(p_ref, w_ref, b_ref, o_ref):
    pv = p_ref[...]
    m = jnp.max(pv, axis=0)
    m = jnp.where(m > jnp.float32(-1e37), m, jnp.float32(0.0))
    m = jnp.where(m >= 0, m, m * jnp.float32(0.01))
    o_ref[...] = (jnp.dot(m, w_ref[...], preferred_element_type=jnp.float32)
                  + b_ref[...])


def _final_call(parts, Wout, bout):
    return pl.pallas_call(
        _final_body,
        out_shape=jax.ShapeDtypeStruct((128, 128), jnp.float32),
    )(parts, Wout, bout.reshape(1, 128))


def _bucket_edges(src, dst):
    g = (dst // NRANGE).astype(jnp.int32)
    gs, src_s, dst_s = lax.sort((g, src, dst), num_keys=1, is_stable=False)
    bsrc = jnp.pad(src_s, (0, 32 * K))
    bdst = jnp.pad(dst_s, (0, 32 * K), constant_values=N)
    off = jnp.searchsorted(gs, jnp.arange(4, dtype=jnp.int32),
                           side="left").astype(jnp.int32)
    cnt = (jnp.concatenate([off[1:], jnp.full((1,), E, jnp.int32)])
           - off).astype(jnp.int32)
    start = (off // K) * K
    span = off + cnt - start
    nch = 2 * ((span + 32 * K - 1) // (32 * K))
    tile = jnp.arange(16, dtype=jnp.int32)
    starts = start[:, None] + tile[None, :] * nch[:, None] * K
    nchs = jnp.broadcast_to(nch[:, None], (4, 16)).astype(jnp.int32)
    pairs = jnp.concatenate(
        [starts.reshape(-1, 1).astype(jnp.int32),
         nchs.reshape(-1, 1),
         jnp.zeros((64, 14), jnp.int32)], axis=1).reshape(-1)
    return bsrc, bdst, pairs


def kernel(x, edge_index, batch_index, alpha,
           W0, as0, ad0, b0, Wr0,
           W1, as1, ad1, b1, Wr1,
           W2, as2, ad2, b2, Wr2,
           W3, as3, ad3, b3, Wr3,
           Wout, bout):
    src = edge_index[0]
    dst = edge_index[1]
    bsrc, bdst, meta = _bucket_edges(src, dst)

    h = jnp.pad(x, ((0, 0), (0, D - F_IN)))
    params = [
        (jnp.pad(W0, ((0, D - F_IN), (0, 0))), as0, ad0, b0,
         jnp.pad(Wr0, ((0, D - F_IN), (0, 0)))),
        (W1, as1, ad1, b1, Wr1),
        (W2, as2, ad2, b2, Wr2),
        (W3, as3, ad3, b3, Wr3),
    ]
    for li, (W, a_s, a_d, b, Wr) in enumerate(params):
        xp, r, av = _tc_layer(h, W, Wr, a_s, a_d, b)
        ast = jnp.pad(av[:, 0], (0, NADP - N))
        adt = jnp.pad(av[:, 1], (0, NADP - N))
        h = _edge_call(li < 3, xp, bsrc, bdst, ast, adt, r, meta)

    parts = _pool_call(h, batch_index, alpha.reshape(-1))
    return _final_call(parts.reshape(32, 128, 128), Wout, bout)
